# Initial kernel scaffold; baseline (speedup 1.0000x reference)
import jax
import jax.numpy as jnp
from jax.experimental import pallas as pl


def kernel(features):
    raise NotImplementedError("write your pallas kernel here")


# speedup vs baseline: 7.1288x; 7.1288x over previous
import functools

import jax
import jax.numpy as jnp
from jax.experimental import pallas as pl

_K = 64
_I32_MIN = -(2 ** 31)
_I32_MAX = 2 ** 31 - 1


def _topk_mask_kernel(x_ref, o_ref, *, k):
    x = x_ref[...]
    b = jax.lax.bitcast_convert_type(x, jnp.int32)
    key = jnp.where(b >= 0, b, b ^ jnp.int32(0x7FFFFFFF))
    rows = x.shape[0]
    lo0 = jnp.full((rows, 1), _I32_MIN, jnp.int32)
    hi0 = jnp.full((rows, 1), _I32_MAX, jnp.int32)

    def body(_, carry):
        lo, hi = carry
        mid = (lo >> 1) + (hi >> 1) + ((lo | hi) & 1)
        cnt = jnp.sum((key >= mid).astype(jnp.int32), axis=-1, keepdims=True)
        ge = cnt >= k
        return jnp.where(ge, mid, lo), jnp.where(ge, hi, mid - 1)

    t, _ = jax.lax.fori_loop(0, 32, body, (lo0, hi0))
    ge = key >= t
    c_ge = jnp.sum(ge.astype(jnp.int32), axis=-1, keepdims=True)
    relu = jnp.maximum(x, 0.0)
    need_fix = jnp.any((c_ge > k) & (t > 0))

    @pl.when(jnp.logical_not(need_fix))
    def _():
        o_ref[...] = jnp.where(ge, relu, 0.0)

    @pl.when(need_fix)
    def _():
        eq = key == t
        c_eq = jnp.sum(eq.astype(jnp.int32), axis=-1, keepdims=True)
        slots = k - (c_ge - c_eq)
        idx = jax.lax.broadcasted_iota(jnp.int32, x.shape, 1)
        lo2 = jnp.zeros((rows, 1), jnp.int32)
        hi2 = jnp.full((rows, 1), x.shape[1] - 1, jnp.int32)

        def body2(_, carry):
            l, h = carry
            m = (l + h) >> 1
            c = jnp.sum((eq & (idx <= m)).astype(jnp.int32), axis=-1,
                        keepdims=True)
            enough = c >= slots
            return jnp.where(enough, l, m + 1), jnp.where(enough, m, h)

        cut, _ = jax.lax.fori_loop(0, 15, body2, (lo2, hi2))
        keep = (key > t) | (eq & (idx <= cut))
        o_ref[...] = jnp.where(keep, relu, 0.0)


def kernel(features):
    B, L, D = features.shape
    x = features.reshape(B * L, D)
    rows_per_block = 8
    out = pl.pallas_call(
        functools.partial(_topk_mask_kernel, k=_K),
        grid=((B * L) // rows_per_block,),
        in_specs=[pl.BlockSpec((rows_per_block, D), lambda i: (i, 0))],
        out_specs=pl.BlockSpec((rows_per_block, D), lambda i: (i, 0)),
        out_shape=jax.ShapeDtypeStruct((B * L, D), jnp.float32),
    )(x)
    return out.reshape(B, L, D)


# speedup vs baseline: 13.6225x; 1.9109x over previous
import functools

import jax
import jax.numpy as jnp
from jax.experimental import pallas as pl

_K = 64
_I32_MIN = -(2 ** 31)
_I32_MAX = 2 ** 31 - 1


def _topk_mask_kernel(x_ref, o_ref, *, k):
    x = x_ref[...]
    b = jax.lax.bitcast_convert_type(x, jnp.int32)
    key = jnp.where(b >= 0, b, b ^ jnp.int32(0x7FFFFFFF))
    rows = x.shape[0]
    lo0 = jnp.full((rows, 1), _I32_MIN, jnp.int32)
    hi0 = jnp.full((rows, 1), _I32_MAX, jnp.int32)

    def body(_, carry):
        lo, hi = carry
        mid = (lo >> 1) + (hi >> 1) + ((lo | hi) & 1)
        cnt = jnp.sum((key >= mid).astype(jnp.int32), axis=-1, keepdims=True)
        ge = cnt >= k
        return jnp.where(ge, mid, lo), jnp.where(ge, hi, mid - 1)

    t, _ = jax.lax.fori_loop(0, 32, body, (lo0, hi0))
    ge = key >= t
    c_ge = jnp.sum(ge.astype(jnp.int32), axis=-1, keepdims=True)
    relu = jnp.maximum(x, 0.0)
    need_fix = jnp.any((c_ge > k) & (t > 0))

    @pl.when(jnp.logical_not(need_fix))
    def _():
        o_ref[...] = jnp.where(ge, relu, 0.0)

    @pl.when(need_fix)
    def _():
        eq = key == t
        c_eq = jnp.sum(eq.astype(jnp.int32), axis=-1, keepdims=True)
        slots = k - (c_ge - c_eq)
        idx = jax.lax.broadcasted_iota(jnp.int32, x.shape, 1)
        lo2 = jnp.zeros((rows, 1), jnp.int32)
        hi2 = jnp.full((rows, 1), x.shape[1] - 1, jnp.int32)

        def body2(_, carry):
            l, h = carry
            m = (l + h) >> 1
            c = jnp.sum((eq & (idx <= m)).astype(jnp.int32), axis=-1,
                        keepdims=True)
            enough = c >= slots
            return jnp.where(enough, l, m + 1), jnp.where(enough, m, h)

        cut, _ = jax.lax.fori_loop(0, 15, body2, (lo2, hi2))
        keep = (key > t) | (eq & (idx <= cut))
        o_ref[...] = jnp.where(keep, relu, 0.0)


def kernel(features):
    B, L, D = features.shape
    x = features.reshape(B * L, D)
    rows_per_block = 64
    out = pl.pallas_call(
        functools.partial(_topk_mask_kernel, k=_K),
        grid=((B * L) // rows_per_block,),
        in_specs=[pl.BlockSpec((rows_per_block, D), lambda i: (i, 0))],
        out_specs=pl.BlockSpec((rows_per_block, D), lambda i: (i, 0)),
        out_shape=jax.ShapeDtypeStruct((B * L, D), jnp.float32),
    )(x)
    return out.reshape(B, L, D)
